# baseline (speedup 1.0000x reference)
import functools

import jax
import jax.numpy as jnp
from jax import lax
from jax.experimental import pallas as pl
from jax.experimental.pallas import tpu as pltpu
from jax.experimental.pallas import tpu_sc as plsc

_NC, _NS = 2, 16
_NW = _NC * _NS


def kernel(rule_prob, token_prob, reference_prob, length):
    L, B, R = rule_prob.shape
    V = token_prob.shape[2]
    M = reference_prob.shape[2]
    tok_t = token_prob.transpose(0, 2, 1)
    ref_t = reference_prob.transpose(0, 2, 1)

    rb_n = B // _NW
    tb_n = 32
    tb_tail = V - 31 * tb_n
    pb_n = 2

    mesh = plsc.VectorSubcoreMesh(core_axis_name="c", subcore_axis_name="s")

    @functools.partial(
        pl.kernel,
        out_type=(
            jax.ShapeDtypeStruct((B, R), jnp.float32),
            jax.ShapeDtypeStruct((V, B), jnp.float32),
            jax.ShapeDtypeStruct((M, B), jnp.float32),
        ),
        mesh=mesh,
        scratch_types=[
            pltpu.VMEM((16,), jnp.int32),
            pltpu.VMEM((rb_n, R), jnp.float32),
            pltpu.VMEM((tb_n, B), jnp.float32),
            pltpu.VMEM((pb_n, B), jnp.float32),
            pltpu.SemaphoreType.DMA,
            pltpu.SemaphoreType.DMA,
            pltpu.SemaphoreType.DMA,
        ],
    )
    def sc_copy(len_hbm, rule_hbm, tok_hbm, ref_hbm, r_out, t_out, p_out,
                len_v, rbuf, tbuf, pbuf, sem_r, sem_t, sem_p):
        w = lax.axis_index("s") * _NC + lax.axis_index("c")
        pltpu.sync_copy(len_hbm, len_v.at[pl.ds(0, 1)])
        idx = len_v[...][0] - 1

        r0 = w * rb_n
        t0 = w * tb_n
        p0 = w * pb_n

        pltpu.async_copy(
            rule_hbm.at[idx, pl.ds(r0, rb_n)], rbuf, sem_r).wait()
        pltpu.async_copy(rbuf, r_out.at[pl.ds(r0, rb_n)], sem_r).wait()

        @pl.when(w < _NW - 1)
        def _():
            pltpu.async_copy(
                tok_hbm.at[idx, pl.ds(t0, tb_n)], tbuf, sem_t).wait()
            pltpu.async_copy(tbuf, t_out.at[pl.ds(t0, tb_n)], sem_t).wait()

        @pl.when(w == _NW - 1)
        def _():
            pltpu.async_copy(
                tok_hbm.at[idx, pl.ds(t0, tb_tail)],
                tbuf.at[pl.ds(0, tb_tail)], sem_t).wait()
            pltpu.async_copy(
                tbuf.at[pl.ds(0, tb_tail)],
                t_out.at[pl.ds(t0, tb_tail)], sem_t).wait()

        @pl.when(w < M // pb_n)
        def _():
            pltpu.async_copy(
                ref_hbm.at[idx, pl.ds(p0, pb_n)], pbuf, sem_p).wait()
            pltpu.async_copy(pbuf, p_out.at[pl.ds(p0, pb_n)], sem_p).wait()

    r, t_t, p_t = sc_copy(length, rule_prob, tok_t, ref_t)
    return (r, t_t.T, p_t.T)


# speedup vs baseline: 4.4190x; 4.4190x over previous
import jax
import jax.numpy as jnp
from jax.experimental import pallas as pl
from jax.experimental.pallas import tpu as pltpu


def _dma3(len_ref, r_ref, t_ref, p_ref, ro_ref, to_ref, po_ref,
          sem_r, sem_t, sem_p):
    idx = len_ref[0] - 1
    cp_t = pltpu.make_async_copy(t_ref.at[idx], to_ref, sem_t)
    cp_r = pltpu.make_async_copy(r_ref.at[idx], ro_ref, sem_r)
    cp_p = pltpu.make_async_copy(p_ref.at[idx], po_ref, sem_p)
    cp_t.start()
    cp_r.start()
    cp_p.start()
    cp_t.wait()
    cp_r.wait()
    cp_p.wait()


def kernel(rule_prob, token_prob, reference_prob, length):
    L, B, R = rule_prob.shape
    V = token_prob.shape[2]
    M = reference_prob.shape[2]
    tok_t = token_prob.transpose(0, 2, 1)
    ref_t = reference_prob.transpose(0, 2, 1)

    r, t_t, p_t = pl.pallas_call(
        _dma3,
        in_specs=[
            pl.BlockSpec(memory_space=pltpu.SMEM),
            pl.BlockSpec(memory_space=pl.ANY),
            pl.BlockSpec(memory_space=pl.ANY),
            pl.BlockSpec(memory_space=pl.ANY),
        ],
        out_specs=[
            pl.BlockSpec((B, R), lambda: (0, 0)),
            pl.BlockSpec((V, B), lambda: (0, 0)),
            pl.BlockSpec((M, B), lambda: (0, 0)),
        ],
        out_shape=[
            jax.ShapeDtypeStruct((B, R), jnp.float32),
            jax.ShapeDtypeStruct((V, B), jnp.float32),
            jax.ShapeDtypeStruct((M, B), jnp.float32),
        ],
        scratch_shapes=[pltpu.SemaphoreType.DMA] * 3,
    )(length, rule_prob, tok_t, ref_t)
    return (r, t_t.T, p_t.T)
